# Initial kernel scaffold; baseline (speedup 1.0000x reference)
import jax
import jax.numpy as jnp
from jax.experimental import pallas as pl


def kernel(edge_index):
    raise NotImplementedError("write your pallas kernel here")


# speedup vs baseline: 143.4462x; 143.4462x over previous
import functools

import jax
import jax.numpy as jnp
from jax import lax
from jax.experimental import pallas as pl
from jax.experimental.pallas import tpu as pltpu
from jax.experimental.pallas import tpu_sc as plsc

NUM_NODES = 50000
NUM_EDGES = 1600000
N_PAD = 50176
NW = 32
E_PER_W = NUM_EDGES // NW
CHUNK = 10000

_mesh = plsc.VectorSubcoreMesh(core_axis_name="c", subcore_axis_name="s")
_sc_params = pltpu.CompilerParams(needs_layout_passes=False)


def _wid():
    return lax.axis_index("s") * 2 + lax.axis_index("c")


@functools.partial(
    pl.kernel,
    mesh=_mesh,
    out_type=jax.ShapeDtypeStruct((NW, N_PAD), jnp.float32),
    compiler_params=_sc_params,
    scratch_types=[
        pltpu.VMEM((E_PER_W,), jnp.int32),
        pltpu.VMEM((N_PAD,), jnp.float32),
    ],
)
def _degree_kernel(col_hbm, deg_out_hbm, idx_v, deg_v):
    wid = _wid()
    pltpu.sync_copy(col_hbm.at[pl.ds(wid * E_PER_W, E_PER_W)], idx_v)

    zeros = jnp.zeros((16,), jnp.float32)

    def zero_body(i, carry):
        deg_v[pl.ds(i * 16, 16)] = zeros
        return carry

    lax.fori_loop(0, N_PAD // 16, zero_body, 0)

    ones = jnp.ones((16,), jnp.float32)

    def add_body(i, carry):
        idx = idx_v[pl.ds(i * 16, 16)]
        plsc.addupdate_scatter(deg_v, [idx], ones)
        return carry

    lax.fori_loop(0, E_PER_W // 16, add_body, 0)

    pltpu.sync_copy(deg_v, deg_out_hbm.at[wid])


def _reduce_rsqrt_body(p_ref, o_ref):
    s = jnp.sum(p_ref[...], axis=0)
    o_ref[...] = jnp.where(s > 0.0, jax.lax.rsqrt(s), 0.0)


@functools.partial(
    pl.kernel,
    mesh=_mesh,
    out_type=jax.ShapeDtypeStruct((NUM_EDGES,), jnp.float32),
    compiler_params=_sc_params,
    scratch_types=[
        pltpu.VMEM((N_PAD,), jnp.float32),
        pltpu.VMEM((CHUNK,), jnp.int32),
        pltpu.VMEM((CHUNK,), jnp.int32),
        pltpu.VMEM((CHUNK,), jnp.float32),
    ],
)
def _norm_kernel(row_hbm, col_hbm, tab_hbm, out_hbm, tab_v, row_v, col_v, out_v):
    wid = _wid()
    base = wid * E_PER_W
    pltpu.sync_copy(tab_hbm, tab_v)

    def chunk_body(ci, carry):
        off = base + ci * CHUNK
        pltpu.sync_copy(row_hbm.at[pl.ds(off, CHUNK)], row_v)
        pltpu.sync_copy(col_hbm.at[pl.ds(off, CHUNK)], col_v)

        def vec_body(i, c2):
            r = plsc.load_gather(tab_v, [row_v[pl.ds(i * 16, 16)]])
            c = plsc.load_gather(tab_v, [col_v[pl.ds(i * 16, 16)]])
            out_v[pl.ds(i * 16, 16)] = r * c
            return c2

        lax.fori_loop(0, CHUNK // 16, vec_body, 0)
        pltpu.sync_copy(out_v, out_hbm.at[pl.ds(off, CHUNK)])
        return carry

    lax.fori_loop(0, E_PER_W // CHUNK, chunk_body, 0)


def kernel(edge_index):
    row = edge_index[0].astype(jnp.int32)
    col = edge_index[1].astype(jnp.int32)
    partials = _degree_kernel(col)
    deg_inv = pl.pallas_call(
        _reduce_rsqrt_body,
        out_shape=jax.ShapeDtypeStruct((392, 128), jnp.float32),
    )(partials.reshape(NW, 392, 128))
    return _norm_kernel(row, col, deg_inv.reshape(N_PAD))


# speedup vs baseline: 199.0426x; 1.3876x over previous
import functools

import jax
import jax.numpy as jnp
from jax import lax
from jax.experimental import pallas as pl
from jax.experimental.pallas import tpu as pltpu
from jax.experimental.pallas import tpu_sc as plsc

NUM_NODES = 50000
NUM_EDGES = 1600000
N_ROWS = 392
NW = 32
E_PER_W = NUM_EDGES // NW
CHUNK = 10000

_mesh = plsc.VectorSubcoreMesh(core_axis_name="c", subcore_axis_name="s")
_sc_params = pltpu.CompilerParams(
    needs_layout_passes=False, use_tc_tiling_on_sc=False
)


def _wid():
    return lax.axis_index("s") * 2 + lax.axis_index("c")


@functools.partial(
    pl.kernel,
    mesh=_mesh,
    out_type=jax.ShapeDtypeStruct((NW, N_ROWS, 128), jnp.float32),
    compiler_params=_sc_params,
    scratch_types=[
        pltpu.VMEM((E_PER_W,), jnp.int32),
        pltpu.VMEM((N_ROWS, 128), jnp.float32),
    ],
)
def _degree_kernel(ei_hbm, deg_out_hbm, idx_v, deg_v):
    wid = _wid()
    pltpu.sync_copy(ei_hbm.at[1, pl.ds(wid * E_PER_W, E_PER_W)], idx_v)

    zeros = jnp.zeros((16,), jnp.float32)

    def zero_body(i, carry):
        deg_v[i // 8, pl.ds((i % 8) * 16, 16)] = zeros
        return carry

    lax.fori_loop(0, N_ROWS * 8, zero_body, 0)

    ones = jnp.ones((16,), jnp.float32)

    def add_body(i, carry):
        idx = idx_v[pl.ds(i * 16, 16)]
        plsc.addupdate_scatter(deg_v, [idx >> 7, idx & 127], ones)
        return carry

    lax.fori_loop(0, E_PER_W // 16, add_body, 0)

    pltpu.sync_copy(deg_v, deg_out_hbm.at[wid])


def _reduce_rsqrt_body(p_ref, o_ref):
    s = jnp.sum(p_ref[...], axis=0)
    o_ref[...] = jnp.where(s > 0.0, jax.lax.rsqrt(s), 0.0)


@functools.partial(
    pl.kernel,
    mesh=_mesh,
    out_type=jax.ShapeDtypeStruct((NUM_EDGES,), jnp.float32),
    compiler_params=_sc_params,
    scratch_types=[
        pltpu.VMEM((N_ROWS, 128), jnp.float32),
        pltpu.VMEM((CHUNK,), jnp.int32),
        pltpu.VMEM((CHUNK,), jnp.int32),
        pltpu.VMEM((CHUNK,), jnp.float32),
    ],
)
def _norm_kernel(ei_hbm, tab_hbm, out_hbm, tab_v, row_v, col_v, out_v):
    wid = _wid()
    base = wid * E_PER_W
    pltpu.sync_copy(tab_hbm, tab_v)

    def chunk_body(ci, carry):
        off = base + ci * CHUNK
        pltpu.sync_copy(ei_hbm.at[0, pl.ds(off, CHUNK)], row_v)
        pltpu.sync_copy(ei_hbm.at[1, pl.ds(off, CHUNK)], col_v)

        def vec_body(i, c2):
            ri = row_v[pl.ds(i * 16, 16)]
            ci2 = col_v[pl.ds(i * 16, 16)]
            r = plsc.load_gather(tab_v, [ri >> 7, ri & 127])
            c = plsc.load_gather(tab_v, [ci2 >> 7, ci2 & 127])
            out_v[pl.ds(i * 16, 16)] = r * c
            return c2

        lax.fori_loop(0, CHUNK // 16, vec_body, 0)
        pltpu.sync_copy(out_v, out_hbm.at[pl.ds(off, CHUNK)])
        return carry

    lax.fori_loop(0, E_PER_W // CHUNK, chunk_body, 0)


def kernel(edge_index):
    ei = edge_index.astype(jnp.int32)
    partials = _degree_kernel(ei)
    deg_inv = pl.pallas_call(
        _reduce_rsqrt_body,
        out_shape=jax.ShapeDtypeStruct((N_ROWS, 128), jnp.float32),
    )(partials)
    return _norm_kernel(ei, deg_inv)


# speedup vs baseline: 316.3870x; 1.5895x over previous
import functools

import jax
import jax.numpy as jnp
from jax import lax
from jax.experimental import pallas as pl
from jax.experimental.pallas import tpu as pltpu
from jax.experimental.pallas import tpu_sc as plsc

NUM_NODES = 50000
NUM_EDGES = 1600000
N_ROWS = 392
NW = 32
E_PER_W = NUM_EDGES // NW
CHUNK = 10000
N_CHUNKS = E_PER_W // CHUNK

_mesh = plsc.VectorSubcoreMesh(core_axis_name="c", subcore_axis_name="s")
_sc_params = pltpu.CompilerParams(
    needs_layout_passes=False, use_tc_tiling_on_sc=False
)


def _wid():
    return lax.axis_index("s") * 2 + lax.axis_index("c")


@functools.partial(
    pl.kernel,
    mesh=_mesh,
    out_type=jax.ShapeDtypeStruct((NW, N_ROWS, 128), jnp.float32),
    compiler_params=_sc_params,
    scratch_types=[
        pltpu.VMEM((E_PER_W,), jnp.int32),
        pltpu.VMEM((N_ROWS, 128), jnp.float32),
        pltpu.SemaphoreType.DMA,
    ],
)
def _degree_kernel(ei_hbm, deg_out_hbm, idx_v, deg_v, sem):
    wid = _wid()
    cp = pltpu.async_copy(ei_hbm.at[1, pl.ds(wid * E_PER_W, E_PER_W)], idx_v, sem)

    zeros = jnp.zeros((16,), jnp.float32)

    @plsc.parallel_loop(0, N_ROWS, unroll=4)
    def _zero(r):
        for c in range(8):
            deg_v[r, pl.ds(c * 16, 16)] = zeros

    cp.wait()

    ones = jnp.ones((16,), jnp.float32)

    @plsc.parallel_loop(0, E_PER_W, 16, unroll=8)
    def _accum(i):
        idx = idx_v[pl.ds(i, 16)]
        plsc.addupdate_scatter(deg_v, [idx >> 7, idx & 127], ones)

    pltpu.sync_copy(deg_v, deg_out_hbm.at[wid])


def _reduce_rsqrt_body(p_ref, o_ref):
    s = jnp.sum(p_ref[...], axis=0)
    o_ref[...] = jnp.where(s > 0.0, jax.lax.rsqrt(s), 0.0)


@functools.partial(
    pl.kernel,
    mesh=_mesh,
    out_type=jax.ShapeDtypeStruct((NUM_EDGES,), jnp.float32),
    compiler_params=_sc_params,
    scratch_types=[
        pltpu.VMEM((N_ROWS, 128), jnp.float32),
        pltpu.VMEM((2, CHUNK), jnp.int32),
        pltpu.VMEM((2, CHUNK), jnp.int32),
        pltpu.VMEM((2, CHUNK), jnp.float32),
        pltpu.SemaphoreType.DMA,
        pltpu.SemaphoreType.DMA,
        pltpu.SemaphoreType.DMA,
    ],
)
def _norm_kernel(
    ei_hbm, tab_hbm, out_hbm, tab_v, row_v, col_v, out_v, sem_tab, sem_in, sem_out
):
    wid = _wid()
    base = wid * E_PER_W

    tab_cp = pltpu.async_copy(tab_hbm, tab_v, sem_tab)

    def start_in(ci, buf):
        off = base + ci * CHUNK
        a = pltpu.async_copy(ei_hbm.at[0, pl.ds(off, CHUNK)], row_v.at[buf], sem_in)
        b = pltpu.async_copy(ei_hbm.at[1, pl.ds(off, CHUNK)], col_v.at[buf], sem_in)
        return a, b

    pending = start_in(0, 0)
    tab_cp.wait()

    out_cp = None
    for ci in range(N_CHUNKS):
        buf = ci % 2
        a, b = pending
        a.wait()
        b.wait()
        if ci + 1 < N_CHUNKS:
            pending = start_in(ci + 1, 1 - buf)

        rbuf = row_v.at[buf]
        cbuf = col_v.at[buf]
        obuf = out_v.at[buf]

        @plsc.parallel_loop(0, CHUNK, 16, unroll=8)
        def _gather(i):
            ri = rbuf[pl.ds(i, 16)]
            ci2 = cbuf[pl.ds(i, 16)]
            r = plsc.load_gather(tab_v, [ri >> 7, ri & 127])
            c = plsc.load_gather(tab_v, [ci2 >> 7, ci2 & 127])
            obuf[pl.ds(i, 16)] = r * c

        if out_cp is not None:
            out_cp.wait()
        out_cp = pltpu.async_copy(
            obuf, out_hbm.at[pl.ds(base + ci * CHUNK, CHUNK)], sem_out
        )
    out_cp.wait()


def kernel(edge_index):
    ei = edge_index.astype(jnp.int32)
    partials = _degree_kernel(ei)
    deg_inv = pl.pallas_call(
        _reduce_rsqrt_body,
        out_shape=jax.ShapeDtypeStruct((N_ROWS, 128), jnp.float32),
    )(partials)
    return _norm_kernel(ei, deg_inv)
